# baseline (speedup 1.0000x reference)
import functools

import jax
import jax.numpy as jnp
from jax import lax
from jax.experimental import pallas as pl
from jax.experimental.pallas import tpu as pltpu
from jax.experimental.pallas import tpu_sc as plsc

_K = 512
_N = 2000000
_R = 2048
_C = 1024
_PAD = _R * _C - _N


def _cumsum_lanes(x, tri):
    return lax.dot_general(x, tri, (((1,), (0,)), ((), ())),
                           preferred_element_type=jnp.float32)


def _cumsum_shift_lanes(x):
    n = x.shape[1]
    d = 1
    while d < n:
        x = x + jnp.concatenate([jnp.zeros((1, d), jnp.float32), x[:, :-d]],
                                axis=1)
        d *= 2
    return x


def _cumsum_rows(x):
    n = x.shape[0]
    d = 1
    while d < n:
        pad = jnp.zeros((d, 1), jnp.float32)
        x = x + jnp.concatenate([pad, x[:-d, :]], axis=0)
        d *= 2
    return x


def _topk_body(grad_ref, vals_ref, gidx_ref, npos_ref):
    g = grad_ref[...]
    b = lax.bitcast_convert_type(g, jnp.int32)
    key = jnp.where(b >= 0, b, b ^ jnp.int32(0x7FFFFFFF))
    npos_ref[...] = jnp.sum((g > 0).astype(jnp.int32)).reshape(1, 1)

    cnt0 = jnp.sum((key >= 0).astype(jnp.int32))
    base0 = jnp.where(cnt0 >= _K, jnp.int32(0), jnp.int32(-2147483648))

    def bs_body(i, base):
        bit = jnp.int32(30) - i
        trial = base + (jnp.int32(1) << bit)
        cnt = jnp.sum((key >= trial).astype(jnp.int32))
        return jnp.where(cnt >= _K, trial, base)

    kstar = lax.fori_loop(0, 31, bs_body, base0)

    mask_gt = key > kstar
    mask_eq = key == kstar
    cnt_gt = jnp.sum(mask_gt.astype(jnp.int32))
    need_eq = (jnp.int32(_K) - cnt_gt).astype(jnp.float32)

    tri = (lax.broadcasted_iota(jnp.int32, (_C, _C), 0)
           <= lax.broadcasted_iota(jnp.int32, (_C, _C), 1)).astype(jnp.float32)
    eqf = mask_eq.astype(jnp.float32)
    eq_incl = _cumsum_lanes(eqf, tri)
    eq_rt = eq_incl[:, -1:]
    eq_ro = _cumsum_rows(eq_rt) - eq_rt
    eq_gex = eq_ro + eq_incl - eqf
    sel = mask_gt | (mask_eq & (eq_gex < need_eq))

    sf = sel.astype(jnp.float32)
    s_incl = _cumsum_lanes(sf, tri)
    w_ex = s_incl - sf

    rsum = s_incl[:, -1:]
    rs_t = jnp.transpose(rsum)
    s_ro_t = _cumsum_shift_lanes(rs_t) - rs_t

    zerof = jnp.float32(0.0)
    jio = lax.broadcasted_iota(jnp.int32, (_K, 1), 0).astype(jnp.float32)
    cmp = s_ro_t <= jio
    rows_f = jnp.sum(cmp.astype(jnp.float32), axis=1, keepdims=True) - 1.0
    base_off = jnp.max(jnp.where(cmp, s_ro_t, jnp.float32(-1.0)),
                       axis=1, keepdims=True)
    lj = jio - base_off

    rb = (lax.broadcasted_iota(jnp.int32, (1, _R), 1).astype(jnp.float32)
          == rows_f).astype(jnp.bfloat16)
    dnr = (((1,), (0,)), ((), ()))

    inv256 = jnp.float32(1.0 / 256.0)
    w_hif = jnp.floor(w_ex * inv256)
    w_hi = jnp.where(sel, w_hif + 1.0, zerof)
    w_lo = jnp.where(sel, w_ex - w_hif * 256.0, zerof)

    def _rgather(mat):
        return lax.dot_general(rb, mat.astype(jnp.bfloat16), dnr,
                               preferred_element_type=jnp.float32)

    gh = _rgather(w_hi)
    gl = _rgather(w_lo)
    g3 = _rgather(((key >> 24) & 255).astype(jnp.float32))
    g2 = _rgather(((key >> 16) & 255).astype(jnp.float32))
    g1 = _rgather(((key >> 8) & 255).astype(jnp.float32))
    g0 = _rgather((key & 255).astype(jnp.float32))
    gkey = ((g3.astype(jnp.int32) << 24) | (g2.astype(jnp.int32) << 16)
            | (g1.astype(jnp.int32) << 8) | g0.astype(jnp.int32))

    gws = jnp.where(gh > 0.5, (gh - 1.0) * 256.0 + gl, jnp.float32(-1.0))
    mrow = gws == lj
    col_iota = lax.broadcasted_iota(jnp.int32, (1, _C), 1).astype(jnp.float32)
    colf = jnp.sum(jnp.where(mrow, col_iota, zerof), axis=1, keepdims=True)
    key_j = jnp.sum(jnp.where(mrow, gkey, jnp.int32(0)), axis=1, keepdims=True)
    b_j = jnp.where(key_j >= 0, key_j, key_j ^ jnp.int32(0x7FFFFFFF))
    v = lax.bitcast_convert_type(b_j, jnp.float32)
    l = rows_f * jnp.float32(_C) + colf
    vT = jnp.transpose(v)
    lT = jnp.transpose(l)
    before = (vT > v) | ((vT == v) & (lT < l))
    rank = jnp.sum(before.astype(jnp.float32), axis=1, keepdims=True)
    perm = rank == lax.broadcasted_iota(jnp.int32, (1, _K), 1).astype(jnp.float32)
    zero = jnp.float32(0.0)
    out_v = jnp.sum(jnp.where(perm, v, zero), axis=0, keepdims=True)
    out_l = jnp.sum(jnp.where(perm, l, zero), axis=0, keepdims=True)
    vals_ref[...] = out_v
    lin = out_l.astype(jnp.int32)
    gidx_ref[:, 0:_K] = lin
    gidx_ref[:, _K:2 * _K] = lin + jnp.int32(_N)


def _run_topk(gpad2d, interpret=False):
    return pl.pallas_call(
        _topk_body,
        out_shape=[
            jax.ShapeDtypeStruct((1, _K), jnp.float32),
            jax.ShapeDtypeStruct((1, 2 * _K), jnp.int32),
            jax.ShapeDtypeStruct((1, 1), jnp.int32),
        ],
        interpret=interpret,
    )(gpad2d)


def _gather_sc(flat_edges, gidx):
    mesh = plsc.VectorSubcoreMesh(core_axis_name="c", subcore_axis_name="s")
    n_per = (2 * _K) // 32

    @functools.partial(
        pl.kernel,
        mesh=mesh,
        out_type=jax.ShapeDtypeStruct((2 * _K,), jnp.int32),
        scratch_types=[
            pltpu.VMEM((n_per,), jnp.int32),
            pltpu.VMEM((n_per,), jnp.int32),
            pltpu.SemaphoreType.DMA,
        ],
    )
    def k(flat_hbm, gidx_hbm, out_hbm, idx_v, g_v, sem):
        wid = lax.axis_index("s") * 2 + lax.axis_index("c")
        base = wid * n_per
        pltpu.sync_copy(gidx_hbm.at[pl.ds(base, n_per)], idx_v)
        pltpu.async_copy(flat_hbm.at[idx_v], g_v, sem).wait()
        pltpu.sync_copy(g_v, out_hbm.at[pl.ds(base, n_per)])

    return k(flat_edges, gidx)


def kernel(gradient, block_edge_index, step_size):
    gpad = jnp.concatenate(
        [gradient, jnp.full((_PAD,), -jnp.inf, jnp.float32)]).reshape(_R, _C)
    vals, gidx, npos = _run_topk(gpad)
    flat = block_edge_index.reshape(-1)
    got = _gather_sc(flat, gidx.reshape(-1))
    flip_edge_index = got.reshape(2, _K)
    scale = jnp.asarray(step_size, jnp.float32) / jnp.float32(_K)
    flip_edge_weight = jnp.ones((_K,), jnp.float32) * scale
    return vals.reshape(_K), flip_edge_index, flip_edge_weight, npos.reshape(())


# speedup vs baseline: 1.0004x; 1.0004x over previous
import functools

import jax
import jax.numpy as jnp
from jax import lax
from jax.experimental import pallas as pl
from jax.experimental.pallas import tpu as pltpu
from jax.experimental.pallas import tpu_sc as plsc

_K = 512
_N = 2000000
_R = 2048
_C = 1024
_PAD = _R * _C - _N


def _cumsum_lanes(x, tri):
    return lax.dot_general(x.astype(jnp.bfloat16), tri, (((1,), (0,)), ((), ())),
                           preferred_element_type=jnp.float32)


def _cumsum_shift_lanes(x):
    n = x.shape[1]
    d = 1
    while d < n:
        x = x + jnp.concatenate([jnp.zeros((1, d), jnp.float32), x[:, :-d]],
                                axis=1)
        d *= 2
    return x


def _cumsum_rows(x):
    n = x.shape[0]
    d = 1
    while d < n:
        pad = jnp.zeros((d, 1), jnp.float32)
        x = x + jnp.concatenate([pad, x[:-d, :]], axis=0)
        d *= 2
    return x


def _topk_body(grad_ref, vals_ref, gidx_ref, npos_ref):
    g = grad_ref[...]
    b = lax.bitcast_convert_type(g, jnp.int32)
    key = jnp.where(b >= 0, b, b ^ jnp.int32(0x7FFFFFFF))
    npos_ref[...] = jnp.sum((g > 0).astype(jnp.int32)).reshape(1, 1)

    cnt0 = jnp.sum((key >= 0).astype(jnp.int32))
    base0 = jnp.where(cnt0 >= _K, jnp.int32(0), jnp.int32(-2147483648))

    def bs_body(i, base):
        bit = jnp.int32(30) - i
        trial = base + (jnp.int32(1) << bit)
        cnt = jnp.sum((key >= trial).astype(jnp.int32))
        return jnp.where(cnt >= _K, trial, base)

    kstar = lax.fori_loop(0, 31, bs_body, base0)

    mask_gt = key > kstar
    mask_eq = key == kstar
    cnt_gt = jnp.sum(mask_gt.astype(jnp.int32))
    need_eq = (jnp.int32(_K) - cnt_gt).astype(jnp.float32)

    tri = (lax.broadcasted_iota(jnp.int32, (_C, _C), 0)
           <= lax.broadcasted_iota(jnp.int32, (_C, _C), 1)).astype(jnp.bfloat16)
    eqf = mask_eq.astype(jnp.float32)
    eq_incl = _cumsum_lanes(eqf, tri)
    eq_rt = eq_incl[:, -1:]
    eq_ro = _cumsum_rows(eq_rt) - eq_rt
    eq_gex = eq_ro + eq_incl - eqf
    sel = mask_gt | (mask_eq & (eq_gex < need_eq))

    sf = sel.astype(jnp.float32)
    s_incl = _cumsum_lanes(sf, tri)
    w_ex = s_incl - sf

    rsum = s_incl[:, -1:]
    rs_t = jnp.transpose(rsum)
    s_ro_t = _cumsum_shift_lanes(rs_t) - rs_t

    zerof = jnp.float32(0.0)
    jio = lax.broadcasted_iota(jnp.int32, (_K, 1), 0).astype(jnp.float32)
    cmp = s_ro_t <= jio
    rows_f = jnp.sum(cmp.astype(jnp.float32), axis=1, keepdims=True) - 1.0
    base_off = jnp.max(jnp.where(cmp, s_ro_t, jnp.float32(-1.0)),
                       axis=1, keepdims=True)
    lj = jio - base_off

    rb = (lax.broadcasted_iota(jnp.int32, (1, _R), 1).astype(jnp.float32)
          == rows_f).astype(jnp.bfloat16)
    dnr = (((1,), (0,)), ((), ()))

    inv256 = jnp.float32(1.0 / 256.0)
    w_hif = jnp.floor(w_ex * inv256)
    w_hi = jnp.where(sel, w_hif + 1.0, zerof)
    w_lo = jnp.where(sel, w_ex - w_hif * 256.0, zerof)

    def _rgather(mat):
        return lax.dot_general(rb, mat.astype(jnp.bfloat16), dnr,
                               preferred_element_type=jnp.float32)

    gh = _rgather(w_hi)
    gl = _rgather(w_lo)
    g3 = _rgather(((key >> 24) & 255).astype(jnp.float32))
    g2 = _rgather(((key >> 16) & 255).astype(jnp.float32))
    g1 = _rgather(((key >> 8) & 255).astype(jnp.float32))
    g0 = _rgather((key & 255).astype(jnp.float32))
    gkey = ((g3.astype(jnp.int32) << 24) | (g2.astype(jnp.int32) << 16)
            | (g1.astype(jnp.int32) << 8) | g0.astype(jnp.int32))

    gws = jnp.where(gh > 0.5, (gh - 1.0) * 256.0 + gl, jnp.float32(-1.0))
    mrow = gws == lj
    col_iota = lax.broadcasted_iota(jnp.int32, (1, _C), 1).astype(jnp.float32)
    colf = jnp.sum(jnp.where(mrow, col_iota, zerof), axis=1, keepdims=True)
    key_j = jnp.sum(jnp.where(mrow, gkey, jnp.int32(0)), axis=1, keepdims=True)
    b_j = jnp.where(key_j >= 0, key_j, key_j ^ jnp.int32(0x7FFFFFFF))
    v = lax.bitcast_convert_type(b_j, jnp.float32)
    l = rows_f * jnp.float32(_C) + colf
    vT = jnp.transpose(v)
    lT = jnp.transpose(l)
    before = (vT > v) | ((vT == v) & (lT < l))
    rank = jnp.sum(before.astype(jnp.float32), axis=1, keepdims=True)
    perm = rank == lax.broadcasted_iota(jnp.int32, (1, _K), 1).astype(jnp.float32)
    zero = jnp.float32(0.0)
    out_v = jnp.sum(jnp.where(perm, v, zero), axis=0, keepdims=True)
    out_l = jnp.sum(jnp.where(perm, l, zero), axis=0, keepdims=True)
    vals_ref[...] = out_v
    lin = out_l.astype(jnp.int32)
    gidx_ref[:, 0:_K] = lin
    gidx_ref[:, _K:2 * _K] = lin + jnp.int32(_N)


def _run_topk(gpad2d, interpret=False):
    return pl.pallas_call(
        _topk_body,
        out_shape=[
            jax.ShapeDtypeStruct((1, _K), jnp.float32),
            jax.ShapeDtypeStruct((1, 2 * _K), jnp.int32),
            jax.ShapeDtypeStruct((1, 1), jnp.int32),
        ],
        interpret=interpret,
    )(gpad2d)


def _gather_sc(flat_edges, gidx):
    mesh = plsc.VectorSubcoreMesh(core_axis_name="c", subcore_axis_name="s")
    n_per = (2 * _K) // 32

    @functools.partial(
        pl.kernel,
        mesh=mesh,
        out_type=jax.ShapeDtypeStruct((2 * _K,), jnp.int32),
        scratch_types=[
            pltpu.VMEM((n_per,), jnp.int32),
            pltpu.VMEM((n_per,), jnp.int32),
            pltpu.SemaphoreType.DMA,
        ],
    )
    def k(flat_hbm, gidx_hbm, out_hbm, idx_v, g_v, sem):
        wid = lax.axis_index("s") * 2 + lax.axis_index("c")
        base = wid * n_per
        pltpu.sync_copy(gidx_hbm.at[pl.ds(base, n_per)], idx_v)
        pltpu.async_copy(flat_hbm.at[idx_v], g_v, sem).wait()
        pltpu.sync_copy(g_v, out_hbm.at[pl.ds(base, n_per)])

    return k(flat_edges, gidx)


def kernel(gradient, block_edge_index, step_size):
    gpad = jnp.concatenate(
        [gradient, jnp.full((_PAD,), -jnp.inf, jnp.float32)]).reshape(_R, _C)
    vals, gidx, npos = _run_topk(gpad)
    flat = block_edge_index.reshape(-1)
    got = _gather_sc(flat, gidx.reshape(-1))
    flip_edge_index = got.reshape(2, _K)
    scale = jnp.asarray(step_size, jnp.float32) / jnp.float32(_K)
    flip_edge_weight = jnp.ones((_K,), jnp.float32) * scale
    return vals.reshape(_K), flip_edge_index, flip_edge_weight, npos.reshape(())
